# baseline (speedup 1.0000x reference)
import functools

import jax
import jax.numpy as jnp
from jax import lax
from jax.experimental import pallas as pl
from jax.experimental.pallas import tpu as pltpu
from jax.experimental.pallas import tpu_sc as plsc

NC = 2
NS = 16
NW = NC * NS
SCH = 2048


@functools.lru_cache(maxsize=None)
def _mp_kernel(e_pad: int, h: int, nt: int, rpw: int, cap: int, bsz: int):
    nchunks = e_pad // SCH
    assert nchunks % 2 == 0 and e_pad % (SCH * 2) == 0
    rpa = -(-(rpw + 1) // 8) * 8
    lcap = cap + 2 * bsz + 16
    nvr = SCH // 16
    shift = {32: 5, 64: 6, 128: 7}[bsz]
    mesh = plsc.VectorSubcoreMesh(core_axis_name="c", subcore_axis_name="s")

    @functools.partial(
        pl.kernel,
        out_type=jax.ShapeDtypeStruct((NC, NS, rpa, h), jnp.float32),
        mesh=mesh,
        compiler_params=pltpu.CompilerParams(needs_layout_passes=False),
        scratch_types=[
            pltpu.VMEM((SCH,), jnp.int32),
            pltpu.VMEM((SCH,), jnp.int32),
            pltpu.VMEM((lcap,), jnp.int32),
            pltpu.VMEM((lcap,), jnp.int32),
            [pltpu.VMEM((bsz,), jnp.int32)] * 2,
            [pltpu.VMEM((bsz, h), jnp.float32)] * 2,
            [pltpu.VMEM((bsz, h), jnp.float32)] * 2,
            pltpu.VMEM((rpa, h), jnp.float32),
            pltpu.SemaphoreType.DMA,
            pltpu.SemaphoreType.DMA,
            [pltpu.SemaphoreType.DMA] * 2,
            [pltpu.SemaphoreType.DMA] * 2,
            [pltpu.SemaphoreType.DMA] * 2,
        ],
    )
    def kern(table, ew, src, mdst, out,
             sb0, sb1, eidb, dlocb, srcv, rows, ewb, acc,
             ssem0, ssem1, esem, tsem, wsem):
        c = lax.axis_index("c")
        s = lax.axis_index("s")
        wid = s * NC + c
        lo = wid * rpw
        iota = lax.iota(jnp.int32, 16)
        zero = jnp.zeros((16,), jnp.float32)

        def zrow(r, carry):
            for hh in range(h // 16):
                acc[r, pl.ds(hh * 16, 16)] = zero
            return carry

        lax.fori_loop(0, rpa, zrow, 0)

        pltpu.async_copy(mdst.at[pl.ds(0, SCH)], sb0, ssem0)

        def do_chunk(ch, scanb, sem, nsb, nsem, carry):
            pltpu.make_async_copy(mdst.at[pl.ds(0, SCH)], scanb, sem).wait()
            pltpu.async_copy(
                mdst.at[pl.ds(((ch + 1) % nchunks) * SCH, SCH)], nsb, nsem)

            def vreg(i, car):
                base, eidvec = car
                dloc = scanb[pl.ds(i * 16, 16)] - lo
                mask = (dloc >= 0) & (dloc < rpw)
                plsc.store_compressed(eidb.at[pl.ds(base, 16)], eidvec,
                                      mask=mask)
                plsc.store_compressed(dlocb.at[pl.ds(base, 16)], dloc,
                                      mask=mask)
                cnt = plsc.all_reduce_population_count(mask)
                return jnp.minimum(base + cnt[0], cap), eidvec + 16

            return lax.fori_loop(0, nvr, vreg, carry)

        def pair(p, carry):
            carry = do_chunk(2 * p, sb0, ssem0, sb1, ssem1, carry)
            carry = do_chunk(2 * p + 1, sb1, ssem1, sb0, ssem0, carry)
            return carry

        base, _ = lax.fori_loop(0, nchunks // 2, pair, (0, iota))
        pltpu.make_async_copy(mdst.at[pl.ds(0, SCH)], sb0, ssem0).wait()

        for t in range(4 * bsz // 16):
            eidb[pl.ds(base + t * 16, 16)] = iota * 0
            dlocb[pl.ds(base + t * 16, 16)] = iota * 0 + rpw

        npairs = (base + (2 * bsz - 1)) >> (shift + 1)

        def compute(b, cur, car):
            off = b * bsz

            def group(g, c2):
                dv = dlocb[pl.ds(off + g * 16, 16)]
                for jj in range(16):
                    j = g * 16 + jj
                    dl = dv[jj]
                    for hh in range(h // 16):
                        v = (ewb[cur][j, pl.ds(hh * 16, 16)]
                             + rows[cur][j, pl.ds(hh * 16, 16)])
                        plsc.addupdate(acc.at[dl, pl.ds(hh * 16, 16)],
                                       jnp.maximum(v, 0.0))
                return c2

            return lax.fori_loop(0, bsz // 16, group, car)

        esl0 = eidb.at[pl.ds(0, bsz)]
        pltpu.sync_copy(src.at[esl0], srcv[0])
        pltpu.async_copy(table.at[srcv[0]], rows[0], tsem[0])
        pltpu.async_copy(ew.at[esl0], ewb[0], wsem[0])
        pltpu.async_copy(src.at[eidb.at[pl.ds(bsz, bsz)]], srcv[1], esem[1])

        def pairloop(p, car):
            for half in (0, 1):
                b = 2 * p + half
                cur, nxt = half, 1 - half
                nsl = eidb.at[pl.ds((b + 1) * bsz, bsz)]
                pltpu.make_async_copy(src.at[nsl], srcv[nxt], esem[nxt]).wait()
                pltpu.async_copy(table.at[srcv[nxt]], rows[nxt], tsem[nxt])
                pltpu.async_copy(ew.at[nsl], ewb[nxt], wsem[nxt])
                pltpu.make_async_copy(table.at[srcv[cur]], rows[cur],
                                      tsem[cur]).wait()
                pltpu.make_async_copy(ew.at[esl0], ewb[cur], wsem[cur]).wait()
                pltpu.async_copy(src.at[eidb.at[pl.ds((b + 2) * bsz, bsz)]],
                                 srcv[cur], esem[cur])
                car = compute(b, cur, car)
            return car

        lax.fori_loop(0, npairs, pairloop, 0)
        pltpu.make_async_copy(table.at[srcv[0]], rows[0], tsem[0]).wait()
        pltpu.make_async_copy(ew.at[esl0], ewb[0], wsem[0]).wait()
        pltpu.make_async_copy(src.at[esl0], srcv[1], esem[1]).wait()
        pltpu.sync_copy(acc, out.at[c, s])

    return kern


@functools.lru_cache(maxsize=None)
def _ew_kernel(e_pad: int, de: int, htot: int, blk: int = 2048):
    def body(ea_ref, w_ref, b_ref, o_ref):
        o_ref[...] = (
            jnp.dot(ea_ref[...], w_ref[...],
                    preferred_element_type=jnp.float32) + b_ref[...])

    return pl.pallas_call(
        body,
        grid=(e_pad // blk,),
        in_specs=[
            pl.BlockSpec((blk, de), lambda i: (i, 0)),
            pl.BlockSpec((de, htot), lambda i: (0, 0)),
            pl.BlockSpec((1, htot), lambda i: (0, 0)),
        ],
        out_specs=pl.BlockSpec((blk, htot), lambda i: (i, 0)),
        out_shape=jax.ShapeDtypeStruct((e_pad, htot), jnp.float32),
    )


def _pad_rows(a, mult=128):
    r = (-a.shape[0]) % mult
    if r == 0:
        return a
    return jnp.concatenate([a, jnp.zeros((r, a.shape[1]), a.dtype)])


def _assemble(out, rpw, nrows):
    arr = out[:, :, :rpw, :]
    arr = arr.transpose(1, 0, 2, 3).reshape(NW * rpw, out.shape[-1])
    return arr[:nrows]


def kernel(x, edge_index, edge_attr, Ws1, Wm1, We1, b1, pool_w,
           Ws2, Wm2, We2, b2, Ws3, Wm3, We3, b3):
    n, f = x.shape
    e = edge_index.shape[1]
    h = Ws1.shape[1]
    out_w = Ws3.shape[1]
    k = n // 2
    neg = jnp.float32(-1e30)

    e_pad = -(-e // (SCH * 2)) * (SCH * 2)
    pad = e_pad - e
    rpw1 = -(-(n + 1) // NW)
    rpw2 = -(-(k + 1) // NW)
    rpw2 = -(-rpw2 // 8) * 8
    cap1, cap2 = 13312, 8192

    src = edge_index[0]
    dst = edge_index[1]
    src_p = jnp.concatenate([src, jnp.zeros((pad,), jnp.int32)])
    dst_p = jnp.concatenate([dst, jnp.full((pad,), n, jnp.int32)])
    ea_p = jnp.concatenate(
        [edge_attr, jnp.zeros((pad, edge_attr.shape[1]), edge_attr.dtype)])
    f_de = edge_attr.shape[1]

    xm1 = x @ Wm1
    ew1 = _ew_kernel(e_pad, f_de, h)(ea_p, We1, b1[None])
    xm1 = _pad_rows(xm1)
    agg1 = _mp_kernel(e_pad, h, xm1.shape[0], rpw1, cap1, 64)(
        xm1, ew1, src_p, dst_p)
    x1 = jax.nn.relu(x @ Ws1 + _assemble(agg1, rpw1, n))

    score = (x1 @ pool_w) / (jnp.linalg.norm(pool_w) + 1e-16)
    vals, perm = jax.lax.top_k(score, k)
    xp = x1[perm] * jnp.tanh(vals)[:, None]
    mapping = jnp.full((n + 1,), -1, dtype=jnp.int32).at[perm].set(
        jnp.arange(k, dtype=jnp.int32))
    msrc_raw = mapping[src_p]
    TR = 512
    spread = k + (jnp.arange(e_pad, dtype=jnp.int32) & (TR - 1))
    msrc = jnp.where(msrc_raw < 0, spread, msrc_raw)
    mdst = mapping[dst_p]

    xm2 = _pad_rows(jnp.concatenate([xp @ Wm2, jnp.full((TR, 2 * h), neg)]))
    ew2 = _ew_kernel(e_pad, f_de, 2 * h)(ea_p, We2, b2[None])
    agg2 = _mp_kernel(e_pad, 2 * h, xm2.shape[0], rpw2, cap2, 32)(
        xm2, ew2, msrc, mdst)
    x2 = jax.nn.relu(xp @ Ws2 + _assemble(agg2, rpw2, k))

    xm3 = _pad_rows(jnp.concatenate([x2 @ Wm3, jnp.full((TR, out_w), neg)]))
    ew3 = _ew_kernel(e_pad, f_de, out_w)(ea_p, We3, b3[None])
    agg3 = _mp_kernel(e_pad, out_w, xm3.shape[0], rpw2, cap2, 64)(
        xm3, ew3, msrc, mdst)
    x3 = jax.nn.relu(x2 @ Ws3 + _assemble(agg3, rpw2, k))
    return x3


# speedup vs baseline: 1.0009x; 1.0009x over previous
import functools

import jax
import jax.numpy as jnp
from jax import lax
from jax.experimental import pallas as pl
from jax.experimental.pallas import tpu as pltpu
from jax.experimental.pallas import tpu_sc as plsc

NC = 2
NS = 16
NW = NC * NS
SCH = 2048


@functools.lru_cache(maxsize=None)
def _mp_kernel(e_pad: int, h: int, nt: int, rpw: int, cap: int, bsz: int):
    nchunks = e_pad // SCH
    assert nchunks % 2 == 0 and e_pad % (SCH * 2) == 0
    rpa = -(-(rpw + 1) // 8) * 8
    lcap = cap + 2 * bsz + 16
    nvr = SCH // 16
    shift = {32: 5, 64: 6, 128: 7}[bsz]
    mesh = plsc.VectorSubcoreMesh(core_axis_name="c", subcore_axis_name="s")

    @functools.partial(
        pl.kernel,
        out_type=jax.ShapeDtypeStruct((NC, NS, rpa, h), jnp.float32),
        mesh=mesh,
        compiler_params=pltpu.CompilerParams(needs_layout_passes=False),
        scratch_types=[
            pltpu.VMEM((SCH,), jnp.int32),
            pltpu.VMEM((SCH,), jnp.int32),
            pltpu.VMEM((lcap,), jnp.int32),
            pltpu.VMEM((lcap,), jnp.int32),
            [pltpu.VMEM((bsz,), jnp.int32)] * 2,
            [pltpu.VMEM((bsz, h), jnp.float32)] * 2,
            [pltpu.VMEM((bsz, h), jnp.float32)] * 2,
            pltpu.VMEM((rpa, h), jnp.float32),
            pltpu.SemaphoreType.DMA,
            pltpu.SemaphoreType.DMA,
            [pltpu.SemaphoreType.DMA] * 2,
            [pltpu.SemaphoreType.DMA] * 2,
            [pltpu.SemaphoreType.DMA] * 2,
        ],
    )
    def kern(table, ew, src, mdst, out,
             sb0, sb1, eidb, dlocb, srcv, rows, ewb, acc,
             ssem0, ssem1, esem, tsem, wsem):
        c = lax.axis_index("c")
        s = lax.axis_index("s")
        wid = s * NC + c
        lo = wid * rpw
        iota = lax.iota(jnp.int32, 16)
        zero = jnp.zeros((16,), jnp.float32)

        def zrow(r, carry):
            for hh in range(h // 16):
                acc[r, pl.ds(hh * 16, 16)] = zero
            return carry

        lax.fori_loop(0, rpa, zrow, 0)

        pltpu.async_copy(mdst.at[pl.ds(0, SCH)], sb0, ssem0)

        def do_chunk(ch, scanb, sem, nsb, nsem, carry):
            pltpu.make_async_copy(mdst.at[pl.ds(0, SCH)], scanb, sem).wait()
            pltpu.async_copy(
                mdst.at[pl.ds(((ch + 1) % nchunks) * SCH, SCH)], nsb, nsem)

            def vreg(i, car):
                base, eidvec = car
                dloc = scanb[pl.ds(i * 16, 16)] - lo
                mask = (dloc >= 0) & (dloc < rpw)
                plsc.store_compressed(eidb.at[pl.ds(base, 16)], eidvec,
                                      mask=mask)
                plsc.store_compressed(dlocb.at[pl.ds(base, 16)], dloc,
                                      mask=mask)
                cnt = plsc.all_reduce_population_count(mask)
                return jnp.minimum(base + cnt[0], cap), eidvec + 16

            return lax.fori_loop(0, nvr, vreg, carry)

        def pair(p, carry):
            carry = do_chunk(2 * p, sb0, ssem0, sb1, ssem1, carry)
            carry = do_chunk(2 * p + 1, sb1, ssem1, sb0, ssem0, carry)
            return carry

        base, _ = lax.fori_loop(0, nchunks // 2, pair, (0, iota))
        pltpu.make_async_copy(mdst.at[pl.ds(0, SCH)], sb0, ssem0).wait()

        for t in range(4 * bsz // 16):
            eidb[pl.ds(base + t * 16, 16)] = iota * 0
            dlocb[pl.ds(base + t * 16, 16)] = iota * 0 + rpw

        npairs = (base + (2 * bsz - 1)) >> (shift + 1)

        def compute(b, cur, car):
            off = b * bsz

            def group(g, c2):
                dv = dlocb[pl.ds(off + g * 16, 16)]
                for jj in range(16):
                    j = g * 16 + jj
                    dl = dv[jj]
                    for hh in range(h // 16):
                        v = (ewb[cur][j, pl.ds(hh * 16, 16)]
                             + rows[cur][j, pl.ds(hh * 16, 16)])
                        plsc.addupdate(acc.at[dl, pl.ds(hh * 16, 16)],
                                       jnp.maximum(v, 0.0))
                return c2

            return lax.fori_loop(0, bsz // 16, group, car)

        esl0 = eidb.at[pl.ds(0, bsz)]
        pltpu.sync_copy(src.at[esl0], srcv[0])
        pltpu.async_copy(table.at[srcv[0]], rows[0], tsem[0])
        pltpu.async_copy(ew.at[esl0], ewb[0], wsem[0])
        pltpu.async_copy(src.at[eidb.at[pl.ds(bsz, bsz)]], srcv[1], esem[1])

        def pairloop(p, car):
            for half in (0, 1):
                b = 2 * p + half
                cur, nxt = half, 1 - half
                nsl = eidb.at[pl.ds((b + 1) * bsz, bsz)]
                pltpu.make_async_copy(src.at[nsl], srcv[nxt], esem[nxt]).wait()
                pltpu.async_copy(table.at[srcv[nxt]], rows[nxt], tsem[nxt])
                pltpu.async_copy(ew.at[nsl], ewb[nxt], wsem[nxt])
                pltpu.make_async_copy(table.at[srcv[cur]], rows[cur],
                                      tsem[cur]).wait()
                pltpu.make_async_copy(ew.at[esl0], ewb[cur], wsem[cur]).wait()
                pltpu.async_copy(src.at[eidb.at[pl.ds((b + 2) * bsz, bsz)]],
                                 srcv[cur], esem[cur])
                car = compute(b, cur, car)
            return car

        lax.fori_loop(0, npairs, pairloop, 0)
        pltpu.make_async_copy(table.at[srcv[0]], rows[0], tsem[0]).wait()
        pltpu.make_async_copy(ew.at[esl0], ewb[0], wsem[0]).wait()
        pltpu.make_async_copy(src.at[esl0], srcv[1], esem[1]).wait()
        pltpu.sync_copy(acc, out.at[c, s])

    return kern


@functools.lru_cache(maxsize=None)
def _ew_kernel(e_pad: int, de: int, htot: int, blk: int = 8192):
    def body(ea_ref, w_ref, b_ref, o_ref):
        o_ref[...] = (
            jnp.dot(ea_ref[...], w_ref[...],
                    preferred_element_type=jnp.float32) + b_ref[...])

    return pl.pallas_call(
        body,
        grid=(e_pad // blk,),
        in_specs=[
            pl.BlockSpec((blk, de), lambda i: (i, 0)),
            pl.BlockSpec((de, htot), lambda i: (0, 0)),
            pl.BlockSpec((1, htot), lambda i: (0, 0)),
        ],
        out_specs=pl.BlockSpec((blk, htot), lambda i: (i, 0)),
        out_shape=jax.ShapeDtypeStruct((e_pad, htot), jnp.float32),
    )


def _pad_rows(a, mult=128):
    r = (-a.shape[0]) % mult
    if r == 0:
        return a
    return jnp.concatenate([a, jnp.zeros((r, a.shape[1]), a.dtype)])


def _assemble(out, rpw, nrows):
    arr = out[:, :, :rpw, :]
    arr = arr.transpose(1, 0, 2, 3).reshape(NW * rpw, out.shape[-1])
    return arr[:nrows]


def kernel(x, edge_index, edge_attr, Ws1, Wm1, We1, b1, pool_w,
           Ws2, Wm2, We2, b2, Ws3, Wm3, We3, b3):
    n, f = x.shape
    e = edge_index.shape[1]
    h = Ws1.shape[1]
    out_w = Ws3.shape[1]
    k = n // 2
    neg = jnp.float32(-1e30)

    e_pad = -(-e // (SCH * 2)) * (SCH * 2)
    pad = e_pad - e
    rpw1 = -(-(n + 1) // NW)
    rpw2 = -(-(k + 1) // NW)
    rpw2 = -(-rpw2 // 8) * 8
    cap1, cap2 = 13312, 8192

    src = edge_index[0]
    dst = edge_index[1]
    src_p = jnp.concatenate([src, jnp.zeros((pad,), jnp.int32)])
    dst_p = jnp.concatenate([dst, jnp.full((pad,), n, jnp.int32)])
    ea_p = jnp.concatenate(
        [edge_attr, jnp.zeros((pad, edge_attr.shape[1]), edge_attr.dtype)])
    f_de = edge_attr.shape[1]

    xm1 = x @ Wm1
    ew1 = _ew_kernel(e_pad, f_de, h)(ea_p, We1, b1[None])
    xm1 = _pad_rows(xm1)
    agg1 = _mp_kernel(e_pad, h, xm1.shape[0], rpw1, cap1, 64)(
        xm1, ew1, src_p, dst_p)
    x1 = jax.nn.relu(x @ Ws1 + _assemble(agg1, rpw1, n))

    score = (x1 @ pool_w) / (jnp.linalg.norm(pool_w) + 1e-16)
    vals, perm = jax.lax.top_k(score, k)
    xp = x1[perm] * jnp.tanh(vals)[:, None]
    mapping = jnp.full((n + 1,), -1, dtype=jnp.int32).at[perm].set(
        jnp.arange(k, dtype=jnp.int32))
    msrc_raw = mapping[src_p]
    TR = 512
    spread = k + (jnp.arange(e_pad, dtype=jnp.int32) & (TR - 1))
    msrc = jnp.where(msrc_raw < 0, spread, msrc_raw)
    mdst = mapping[dst_p]

    xm2 = _pad_rows(jnp.concatenate([xp @ Wm2, jnp.full((TR, 2 * h), neg)]))
    ew2 = _ew_kernel(e_pad, f_de, 2 * h)(ea_p, We2, b2[None])
    agg2 = _mp_kernel(e_pad, 2 * h, xm2.shape[0], rpw2, cap2, 32)(
        xm2, ew2, msrc, mdst)
    x2 = jax.nn.relu(xp @ Ws2 + _assemble(agg2, rpw2, k))

    xm3 = _pad_rows(jnp.concatenate([x2 @ Wm3, jnp.full((TR, out_w), neg)]))
    ew3 = _ew_kernel(e_pad, f_de, out_w)(ea_p, We3, b3[None])
    agg3 = _mp_kernel(e_pad, out_w, xm3.shape[0], rpw2, cap2, 64)(
        xm3, ew3, msrc, mdst)
    x3 = jax.nn.relu(x2 @ Ws3 + _assemble(agg3, rpw2, k))
    return x3


# speedup vs baseline: 2.7474x; 2.7450x over previous
import functools

import jax
import jax.numpy as jnp
from jax import lax
from jax.experimental import pallas as pl
from jax.experimental.pallas import tpu as pltpu
from jax.experimental.pallas import tpu_sc as plsc

NC = 2
NS = 16
NW = NC * NS
SCH = 2048


@functools.lru_cache(maxsize=None)
def _mp_kernel(e_pad: int, h: int, nt: int, rpw: int, cap: int, bsz: int,
               nmap: int = 0):
    nchunks = e_pad // SCH
    assert nchunks % 2 == 0 and e_pad % (SCH * 2) == 0
    rpa = -(-(rpw + 1) // 8) * 8
    lcap = cap + 2 * bsz + 16
    nvr = SCH // 16
    shift = {32: 5, 64: 6, 128: 7}[bsz]
    mesh = plsc.VectorSubcoreMesh(core_axis_name="c", subcore_axis_name="s")

    @functools.partial(
        pl.kernel,
        out_type=jax.ShapeDtypeStruct((NC, NS, rpa, h), jnp.float32),
        mesh=mesh,
        compiler_params=pltpu.CompilerParams(needs_layout_passes=False),
        scratch_types=[
            pltpu.VMEM((SCH,), jnp.int32),
            pltpu.VMEM((SCH,), jnp.int32),
            pltpu.VMEM((lcap,), jnp.int32),
            pltpu.VMEM((lcap,), jnp.int32),
            [pltpu.VMEM((bsz,), jnp.int32)] * 2,
            [pltpu.VMEM((bsz, h), jnp.float32)] * 2,
            [pltpu.VMEM((bsz, h), jnp.float32)] * 2,
            pltpu.VMEM((rpa, h), jnp.float32),
            [pltpu.VMEM((max(nmap, 16),), jnp.int32)] * 2,
            pltpu.SemaphoreType.DMA,
            pltpu.SemaphoreType.DMA,
            [pltpu.SemaphoreType.DMA] * 2,
            [pltpu.SemaphoreType.DMA] * 2,
            [pltpu.SemaphoreType.DMA] * 2,
        ],
    )
    def kern(table, ew, src, mdst, *rest):
        if nmap:
            smap_h, dmap_h, out = rest[:3]
            rest = rest[3:]
        else:
            out = rest[0]
            rest = rest[1:]
        (sb0, sb1, eidb, dlocb, srcv, rows, ewb, acc, maps,
         ssem0, ssem1, esem, tsem, wsem) = rest
        smapv, dmapv = maps
        c = lax.axis_index("c")
        s = lax.axis_index("s")
        wid = s * NC + c
        lo = wid * rpw
        iota = lax.iota(jnp.int32, 16)
        zero = jnp.zeros((16,), jnp.float32)
        if nmap:
            pltpu.sync_copy(smap_h, smapv)
            pltpu.sync_copy(dmap_h, dmapv)

        def zrow(r, carry):
            for hh in range(h // 16):
                acc[r, pl.ds(hh * 16, 16)] = zero
            return carry

        lax.fori_loop(0, rpa, zrow, 0)

        pltpu.async_copy(mdst.at[pl.ds(0, SCH)], sb0, ssem0)

        def do_chunk(ch, scanb, sem, nsb, nsem, carry):
            pltpu.make_async_copy(mdst.at[pl.ds(0, SCH)], scanb, sem).wait()
            pltpu.async_copy(
                mdst.at[pl.ds(((ch + 1) % nchunks) * SCH, SCH)], nsb, nsem)

            def vreg(i, car):
                base, eidvec = car
                dvec = scanb[pl.ds(i * 16, 16)]
                if nmap:
                    dvec = plsc.load_gather(dmapv, [dvec])
                dloc = dvec - lo
                mask = (dloc >= 0) & (dloc < rpw)
                plsc.store_compressed(eidb.at[pl.ds(base, 16)], eidvec,
                                      mask=mask)
                plsc.store_compressed(dlocb.at[pl.ds(base, 16)], dloc,
                                      mask=mask)
                cnt = plsc.all_reduce_population_count(mask)
                return jnp.minimum(base + cnt[0], cap), eidvec + 16

            return lax.fori_loop(0, nvr, vreg, carry)

        def pair(p, carry):
            carry = do_chunk(2 * p, sb0, ssem0, sb1, ssem1, carry)
            carry = do_chunk(2 * p + 1, sb1, ssem1, sb0, ssem0, carry)
            return carry

        base, _ = lax.fori_loop(0, nchunks // 2, pair, (0, iota))
        pltpu.make_async_copy(mdst.at[pl.ds(0, SCH)], sb0, ssem0).wait()

        for t in range(4 * bsz // 16):
            eidb[pl.ds(base + t * 16, 16)] = iota * 0
            dlocb[pl.ds(base + t * 16, 16)] = iota * 0 + rpw

        npairs = (base + (2 * bsz - 1)) >> (shift + 1)

        def compute(b, cur, car):
            off = b * bsz

            def group(g, c2):
                dv = dlocb[pl.ds(off + g * 16, 16)]
                for jj in range(16):
                    j = g * 16 + jj
                    dl = dv[jj]
                    for hh in range(h // 16):
                        v = (ewb[cur][j, pl.ds(hh * 16, 16)]
                             + rows[cur][j, pl.ds(hh * 16, 16)])
                        plsc.addupdate(acc.at[dl, pl.ds(hh * 16, 16)],
                                       jnp.maximum(v, 0.0))
                return c2

            return lax.fori_loop(0, bsz // 16, group, car)

        def remap_src(buf):
            if nmap:
                for g in range(bsz // 16):
                    sv = plsc.load_gather(smapv, [buf[pl.ds(g * 16, 16)]])
                    buf[pl.ds(g * 16, 16)] = sv

        esl0 = eidb.at[pl.ds(0, bsz)]
        pltpu.sync_copy(src.at[esl0], srcv[0])
        remap_src(srcv[0])
        pltpu.async_copy(table.at[srcv[0]], rows[0], tsem[0])
        pltpu.async_copy(ew.at[esl0], ewb[0], wsem[0])
        pltpu.async_copy(src.at[eidb.at[pl.ds(bsz, bsz)]], srcv[1], esem[1])

        def pairloop(p, car):
            for half in (0, 1):
                b = 2 * p + half
                cur, nxt = half, 1 - half
                nsl = eidb.at[pl.ds((b + 1) * bsz, bsz)]
                pltpu.make_async_copy(src.at[nsl], srcv[nxt], esem[nxt]).wait()
                remap_src(srcv[nxt])
                pltpu.async_copy(table.at[srcv[nxt]], rows[nxt], tsem[nxt])
                pltpu.async_copy(ew.at[nsl], ewb[nxt], wsem[nxt])
                pltpu.make_async_copy(table.at[srcv[cur]], rows[cur],
                                      tsem[cur]).wait()
                pltpu.make_async_copy(ew.at[esl0], ewb[cur], wsem[cur]).wait()
                pltpu.async_copy(src.at[eidb.at[pl.ds((b + 2) * bsz, bsz)]],
                                 srcv[cur], esem[cur])
                car = compute(b, cur, car)
            return car

        lax.fori_loop(0, npairs, pairloop, 0)
        pltpu.make_async_copy(table.at[srcv[0]], rows[0], tsem[0]).wait()
        pltpu.make_async_copy(ew.at[esl0], ewb[0], wsem[0]).wait()
        pltpu.make_async_copy(src.at[esl0], srcv[1], esem[1]).wait()
        pltpu.sync_copy(acc, out.at[c, s])

    return kern


@functools.lru_cache(maxsize=None)
def _ew_kernel(e_pad: int, de: int, htot: int, blk: int = 4096):
    assert e_pad % blk == 0

    def body(ea_ref, w_ref, b_ref, o_ref):
        o_ref[...] = (
            jnp.dot(ea_ref[...], w_ref[...],
                    preferred_element_type=jnp.float32) + b_ref[...])

    return pl.pallas_call(
        body,
        grid=(e_pad // blk,),
        in_specs=[
            pl.BlockSpec((blk, de), lambda i: (i, 0)),
            pl.BlockSpec((de, htot), lambda i: (0, 0)),
            pl.BlockSpec((1, htot), lambda i: (0, 0)),
        ],
        out_specs=pl.BlockSpec((blk, htot), lambda i: (i, 0)),
        out_shape=jax.ShapeDtypeStruct((e_pad, htot), jnp.float32),
    )


def _pad_rows(a, mult=128):
    r = (-a.shape[0]) % mult
    if r == 0:
        return a
    return jnp.concatenate([a, jnp.zeros((r, a.shape[1]), a.dtype)])


def _assemble(out, rpw, nrows):
    arr = out[:, :, :rpw, :]
    arr = arr.transpose(1, 0, 2, 3).reshape(NW * rpw, out.shape[-1])
    return arr[:nrows]


def kernel(x, edge_index, edge_attr, Ws1, Wm1, We1, b1, pool_w,
           Ws2, Wm2, We2, b2, Ws3, Wm3, We3, b3):
    n, f = x.shape
    e = edge_index.shape[1]
    h = Ws1.shape[1]
    out_w = Ws3.shape[1]
    k = n // 2
    neg = jnp.float32(-1e30)

    e_pad = -(-e // (SCH * 2)) * (SCH * 2)
    pad = e_pad - e
    rpw1 = -(-(n + 1) // NW)
    rpw2 = -(-(k + 1) // NW)
    rpw2 = -(-rpw2 // 8) * 8
    cap1, cap2 = 13312, 8192

    src = edge_index[0]
    dst = edge_index[1]
    src_p = jnp.concatenate([src, jnp.zeros((pad,), jnp.int32)])
    dst_p = jnp.concatenate([dst, jnp.full((pad,), n, jnp.int32)])
    ea_p = jnp.concatenate(
        [edge_attr, jnp.zeros((pad, edge_attr.shape[1]), edge_attr.dtype)])
    f_de = edge_attr.shape[1]

    xm1 = x @ Wm1
    ew1 = _ew_kernel(e_pad, f_de, h)(ea_p, We1, b1[None])
    xm1 = _pad_rows(xm1)
    agg1 = _mp_kernel(e_pad, h, xm1.shape[0], rpw1, cap1, 64)(
        xm1, ew1, src_p, dst_p)
    x1 = jax.nn.relu(x @ Ws1 + _assemble(agg1, rpw1, n))

    score = (x1 @ pool_w) / (jnp.linalg.norm(pool_w) + 1e-16)
    vals, perm = jax.lax.top_k(score, k)
    xp = x1[perm] * jnp.tanh(vals)[:, None]
    nmap = n + 16
    dmap = jnp.full((nmap,), -1, dtype=jnp.int32).at[perm].set(
        jnp.arange(k, dtype=jnp.int32))
    TR = 512
    smap = jnp.where(dmap < 0, k + (jnp.arange(nmap, dtype=jnp.int32) & (TR - 1)),
                     dmap)

    xm2 = _pad_rows(jnp.concatenate([xp @ Wm2, jnp.full((TR, 2 * h), neg)]))
    ew2 = _ew_kernel(e_pad, f_de, 2 * h)(ea_p, We2, b2[None])
    agg2 = _mp_kernel(e_pad, 2 * h, xm2.shape[0], rpw2, cap2, 32, nmap)(
        xm2, ew2, src_p, dst_p, smap, dmap)
    x2 = jax.nn.relu(xp @ Ws2 + _assemble(agg2, rpw2, k))

    xm3 = _pad_rows(jnp.concatenate([x2 @ Wm3, jnp.full((TR, out_w), neg)]))
    ew3 = _ew_kernel(e_pad, f_de, out_w)(ea_p, We3, b3[None])
    agg3 = _mp_kernel(e_pad, out_w, xm3.shape[0], rpw2, cap2, 64, nmap)(
        xm3, ew3, src_p, dst_p, smap, dmap)
    x3 = jax.nn.relu(x2 @ Ws3 + _assemble(agg3, rpw2, k))
    return x3
